# baseline (speedup 1.0000x reference)
import functools

import jax
import jax.numpy as jnp
from jax import lax
from jax.experimental import pallas as pl
from jax.experimental.pallas import tpu as pltpu
from jax.experimental.pallas import tpu_sc as plsc

N = 10000
E = 320000
D = 128
G = 10
BN = 1000

NS = 16
NC = 2
NW = NC * NS
K = 128
EPT = E // NW
NCF = 80
EPP = NCF * K
NP = 10240
NPASS = 8
NPH = NP // NPASS
TR = 128
NPA = NPH + TR
NPZ = NPA // NS
NPE = NPH // NS


@functools.cache
def _sc_kernels():
    mesh = plsc.VectorSubcoreMesh(core_axis_name="c", subcore_axis_name="s",
                                  num_cores=NC, num_subcores=NS)

    def _mk(with_counts):
        out_type = [jax.ShapeDtypeStruct((NC * NP, D), jnp.float32)]
        scratch = [
            pltpu.VMEM((NCF, K), jnp.int32),
            pltpu.VMEM((NCF, K), jnp.int32),
            pltpu.VMEM((K, D), jnp.float32),
            pltpu.VMEM((K, D), jnp.float32),
            pltpu.VMEM((NPZ, D), jnp.float32),
            pltpu.VMEM_SHARED((NPA, D), jnp.float32),
        ]
        if with_counts:
            out_type.append(jax.ShapeDtypeStruct((NC * NP, D), jnp.float32))
            scratch.append(pltpu.VMEM((K, D), jnp.float32))
            scratch.append(pltpu.VMEM_SHARED((NPA, D), jnp.float32))
        scratch.append(pltpu.SemaphoreType.DMA)
        scratch.append(pltpu.SemaphoreType.DMA)

        @functools.partial(pl.kernel, out_type=tuple(out_type), mesh=mesh,
                           scratch_types=tuple(scratch))
        def sc_featsum(hr_hbm, src_hbm, dst_hbm, z_hbm, ones_hbm, *rest):
            if with_counts:
                (agg_out, cnt_out, src_v, dst_v, rows_a, rows_b, z_v, agg_sh,
                 ones_v, cnt_sh, sem_a, sem_b) = rest
            else:
                (agg_out, src_v, dst_v, rows_a, rows_b, z_v, agg_sh,
                 sem_a, sem_b) = rest
            cid = lax.axis_index("c")
            sid = lax.axis_index("s")
            wid = cid * NS + sid
            pltpu.sync_copy(src_hbm.at[wid], src_v)
            pltpu.sync_copy(z_hbm, z_v)
            if with_counts:
                pltpu.sync_copy(ones_hbm, ones_v)
            for p in range(NPASS):
                pltpu.sync_copy(z_v, agg_sh.at[pl.ds(sid * NPZ, NPZ)])
                if with_counts:
                    pltpu.sync_copy(z_v, cnt_sh.at[pl.ds(sid * NPZ, NPZ)])
                pltpu.sync_copy(dst_hbm.at[p * NW + wid], dst_v)
                plsc.subcore_barrier()

                def _drain(buf, sem, c):
                    pltpu.make_async_copy(hr_hbm.at[src_v.at[c]], buf,
                                          sem).wait()
                    pltpu.sync_copy(buf, agg_sh.at[dst_v.at[c]], add=True)
                    if with_counts:
                        pltpu.sync_copy(ones_v, cnt_sh.at[dst_v.at[c]],
                                        add=True)

                pltpu.async_copy(hr_hbm.at[src_v.at[0]], rows_a, sem_a)
                pltpu.async_copy(hr_hbm.at[src_v.at[1]], rows_b, sem_b)

                def body(jj, carry):
                    j = 2 * jj
                    _drain(rows_a, sem_a, j)
                    pltpu.async_copy(hr_hbm.at[src_v.at[j + 2]], rows_a, sem_a)
                    _drain(rows_b, sem_b, j + 1)
                    pltpu.async_copy(hr_hbm.at[src_v.at[j + 3]], rows_b, sem_b)
                    return carry

                lax.fori_loop(0, NCF // 2 - 1, body, 0)
                _drain(rows_a, sem_a, NCF - 2)
                _drain(rows_b, sem_b, NCF - 1)
                plsc.subcore_barrier()
                pltpu.sync_copy(agg_sh.at[pl.ds(sid * NPE, NPE)],
                                z_v.at[pl.ds(0, NPE)])
                pltpu.sync_copy(
                    z_v.at[pl.ds(0, NPE)],
                    agg_out.at[pl.ds(cid * NP + p * NPH + sid * NPE, NPE)])
                if with_counts:
                    pltpu.sync_copy(cnt_sh.at[pl.ds(sid * NPE, NPE)],
                                    z_v.at[pl.ds(0, NPE)])
                    pltpu.sync_copy(
                        z_v.at[pl.ds(0, NPE)],
                        cnt_out.at[pl.ds(cid * NP + p * NPH + sid * NPE,
                                         NPE)])
                if p < NPASS - 1:
                    pltpu.sync_copy(z_hbm, z_v)
                    plsc.subcore_barrier()

        return sc_featsum

    return _mk(True), _mk(False)


def _ln_relu(h, g, b):
    mu = jnp.mean(h, axis=-1, keepdims=True)
    xc = h - mu
    var = jnp.mean(xc * xc, axis=-1, keepdims=True)
    hn = xc / jnp.sqrt(var + 1e-5) * g + b
    return jnp.maximum(hn, 0.0)


def _tc_in_body(x_ref, w_ref, b_ref, g_ref, bb_ref, h_ref, hr_ref):
    h = jnp.dot(x_ref[...], w_ref[...], preferred_element_type=jnp.float32)
    h = h + b_ref[...]
    h_ref[...] = h
    hr_ref[...] = _ln_relu(h, g_ref[...], bb_ref[...])


def _mlp_update(h_ref, hr_ref, sa_ref, sb_ref, ca_ref, cb_ref,
                w1_ref, b1_ref, w2_ref, b2_ref):
    cnt = ca_ref[0][:, :1] + cb_ref[0][:, :1]
    s = sa_ref[0] + sb_ref[0]
    agg = (s + 1e-7 * cnt) / jnp.maximum(cnt, 1.0)
    u = hr_ref[...] + agg
    m = jnp.maximum(
        jnp.dot(u, w1_ref[...], preferred_element_type=jnp.float32)
        + b1_ref[...], 0.0)
    m = jnp.dot(m, w2_ref[...], preferred_element_type=jnp.float32) + b2_ref[...]
    return h_ref[...] + m


def _tc_mid_body(h_ref, hr_ref, sa_ref, sb_ref, ca_ref, cb_ref,
                 w1_ref, b1_ref, w2_ref, b2_ref,
                 g_ref, bb_ref, h1_ref, hr1_ref):
    h1 = _mlp_update(h_ref, hr_ref, sa_ref, sb_ref, ca_ref, cb_ref,
                     w1_ref, b1_ref, w2_ref, b2_ref)
    h1_ref[...] = h1
    hr1_ref[...] = _ln_relu(h1, g_ref[...], bb_ref[...])


def _tc_pool_body(h_ref, out_ref):
    out_ref[...] = (jnp.sum(h_ref[...], axis=0, keepdims=True) * (1.0 / BN))[None]


def _blk(shape, index_map):
    return pl.BlockSpec(shape, index_map)


_full0 = lambda i: (0, 0)
_rows = lambda i: (i, 0)

_tc_in = pl.pallas_call(
    _tc_in_body,
    grid=(G,),
    in_specs=[
        _blk((BN, D), _rows),
        _blk((D, D), _full0),
        _blk((1, D), _full0),
        _blk((1, D), _full0),
        _blk((1, D), _full0),
    ],
    out_specs=[_blk((BN, D), _rows), _blk((BN, D), _rows)],
    out_shape=[jax.ShapeDtypeStruct((N, D), jnp.float32)] * 2,
)

_sc0 = lambda i: (0, i, 0)
_sc1 = lambda i: (1, i, 0)

_mid_in_specs = [
    _blk((BN, D), _rows),
    _blk((BN, D), _rows),
    _blk((1, BN, D), _sc0),
    _blk((1, BN, D), _sc1),
    _blk((1, BN, D), _sc0),
    _blk((1, BN, D), _sc1),
    _blk((D, 2 * D), _full0),
    _blk((1, 2 * D), _full0),
    _blk((2 * D, D), _full0),
    _blk((1, D), _full0),
]

_tc_mid = pl.pallas_call(
    _tc_mid_body,
    grid=(G,),
    in_specs=_mid_in_specs + [_blk((1, D), _full0), _blk((1, D), _full0)],
    out_specs=[_blk((BN, D), _rows), _blk((BN, D), _rows)],
    out_shape=[jax.ShapeDtypeStruct((N, D), jnp.float32)] * 2,
)

_tc_pool = pl.pallas_call(
    _tc_pool_body,
    grid=(G,),
    in_specs=[_blk((BN, D), _rows)],
    out_specs=_blk((1, 1, D), lambda i: (i, 0, 0)),
    out_shape=jax.ShapeDtypeStruct((G, 1, D), jnp.float32),
)


def kernel(x, edge_index, num_graphs, W_in, b_in, Wm1_0, bm1_0, Wm2_0, bm2_0,
           ln_g_0, ln_b_0, Wm1_1, bm1_1, Wm2_1, bm2_1, ln_g_1, ln_b_1):
    pad = ((0, 0), (0, EPP - EPT))
    src_f = jnp.pad(edge_index[0].reshape(NW, EPT), pad).reshape(NW, NCF, K)
    dst = jnp.pad(edge_index[1].reshape(NW, EPT), pad,
                  constant_values=-1).reshape(1, NW * EPP)
    p_arr = jnp.arange(NPASS, dtype=jnp.int32)[:, None]
    dl = dst - p_arr * NPH
    ok = (dl >= 0) & (dl < NPH)
    dst_f = jnp.where(ok, dl, NPH + (dst & (TR - 1)))
    dst_f = dst_f.reshape(NPASS * NW, NCF, K)
    z_d = jnp.zeros((NPZ, D), jnp.float32)
    ones = jnp.ones((K, D), jnp.float32)

    sc_featsum_c, sc_featsum_n = _sc_kernels()
    h0, hr0 = _tc_in(x, W_in, b_in.reshape(1, D), ln_g_0.reshape(1, D),
                     ln_b_0.reshape(1, D))
    S0, C0 = sc_featsum_c(hr0, src_f, dst_f, z_d, ones)
    S0, C0 = S0.reshape(NC, NP, D), C0.reshape(NC, NP, D)
    h1, hr1 = _tc_mid(h0, hr0, S0, S0, C0, C0,
                      Wm1_0, bm1_0.reshape(1, 2 * D), Wm2_0,
                      bm2_0.reshape(1, D), ln_g_1.reshape(1, D),
                      ln_b_1.reshape(1, D))
    (S1,) = sc_featsum_n(hr1, src_f, dst_f, z_d, ones)
    S1 = S1.reshape(NC, NP, D)
    h2, _ = _tc_mid(h1, hr1, S1, S1, C0, C0,
                    Wm1_1, bm1_1.reshape(1, 2 * D), Wm2_1,
                    bm2_1.reshape(1, D), ln_g_1.reshape(1, D),
                    ln_b_1.reshape(1, D))
    return _tc_pool(h2).reshape(G, D)


# speedup vs baseline: 1.6364x; 1.6364x over previous
import functools

import jax
import jax.numpy as jnp
from jax import lax
from jax.experimental import pallas as pl
from jax.experimental.pallas import tpu as pltpu
from jax.experimental.pallas import tpu_sc as plsc

N = 10000
E = 320000
D = 128
G = 10
BN = 1000

NS = 16
NC = 2
NW = NC * NS
K = 128
EPT = E // NW
NCF = -(-EPT // K)
EPP = NCF * K
NP = 10240
NPASS_C = 8
NPASS_N = 4
TR = 128


@functools.cache
def _sc_kernels():
    mesh = plsc.VectorSubcoreMesh(core_axis_name="c", subcore_axis_name="s",
                                  num_cores=NC, num_subcores=NS)

    def _mk(with_counts, npass):
        nph = NP // npass
        npa = nph + TR
        npz = npa // NS
        npe = nph // NS
        out_type = [jax.ShapeDtypeStruct((NC * NP, D), jnp.float32)]
        scratch = [
            pltpu.VMEM((NCF, K), jnp.int32),
            pltpu.VMEM((NCF, K), jnp.int32),
            pltpu.VMEM((K, D), jnp.float32),
            pltpu.VMEM((npz, D), jnp.float32),
            pltpu.VMEM_SHARED((npa, D), jnp.float32),
        ]
        if with_counts:
            out_type.append(jax.ShapeDtypeStruct((NC * NP, D), jnp.float32))
            scratch.append(pltpu.VMEM((K, D), jnp.float32))
            scratch.append(pltpu.VMEM_SHARED((npa, D), jnp.float32))
        scratch.append(pltpu.SemaphoreType.DMA)

        @functools.partial(pl.kernel, out_type=tuple(out_type), mesh=mesh,
                           scratch_types=tuple(scratch))
        def sc_featsum(hr_hbm, src_hbm, dst_hbm, z_hbm, ones_hbm, *rest):
            if with_counts:
                (agg_out, cnt_out, src_v, dst_v, rows_v, z_v, agg_sh,
                 ones_v, cnt_sh, sem) = rest
            else:
                agg_out, src_v, dst_v, rows_v, z_v, agg_sh, sem = rest
            cid = lax.axis_index("c")
            sid = lax.axis_index("s")
            wid = cid * NS + sid
            pltpu.sync_copy(src_hbm.at[wid], src_v)
            pltpu.sync_copy(z_hbm, z_v)
            if with_counts:
                pltpu.sync_copy(ones_hbm, ones_v)
            for p in range(npass):
                pltpu.sync_copy(z_v, agg_sh.at[pl.ds(sid * npz, npz)])
                if with_counts:
                    pltpu.sync_copy(z_v, cnt_sh.at[pl.ds(sid * npz, npz)])
                pltpu.sync_copy(dst_hbm.at[p * NW + wid], dst_v)
                plsc.subcore_barrier()

                def body(j, carry):
                    cp = pltpu.async_copy(hr_hbm.at[src_v.at[j]], rows_v, sem)
                    cp.wait()
                    pltpu.sync_copy(rows_v, agg_sh.at[dst_v.at[j]], add=True)
                    if with_counts:
                        pltpu.sync_copy(ones_v, cnt_sh.at[dst_v.at[j]],
                                        add=True)
                    return carry

                lax.fori_loop(0, NCF, body, 0)
                plsc.subcore_barrier()
                pltpu.sync_copy(agg_sh.at[pl.ds(sid * npe, npe)],
                                z_v.at[pl.ds(0, npe)])
                pltpu.sync_copy(
                    z_v.at[pl.ds(0, npe)],
                    agg_out.at[pl.ds(cid * NP + p * nph + sid * npe, npe)])
                if with_counts:
                    pltpu.sync_copy(cnt_sh.at[pl.ds(sid * npe, npe)],
                                    z_v.at[pl.ds(0, npe)])
                    pltpu.sync_copy(
                        z_v.at[pl.ds(0, npe)],
                        cnt_out.at[pl.ds(cid * NP + p * nph + sid * npe,
                                         npe)])
                if p < npass - 1:
                    pltpu.sync_copy(z_hbm, z_v)
                    plsc.subcore_barrier()

        return sc_featsum

    return _mk(True, NPASS_C), _mk(False, NPASS_N)


def _ln_relu(h, g, b):
    mu = jnp.mean(h, axis=-1, keepdims=True)
    xc = h - mu
    var = jnp.mean(xc * xc, axis=-1, keepdims=True)
    hn = xc / jnp.sqrt(var + 1e-5) * g + b
    return jnp.maximum(hn, 0.0)


def _tc_in_body(x_ref, w_ref, b_ref, g_ref, bb_ref, h_ref, hr_ref):
    h = jnp.dot(x_ref[...], w_ref[...], preferred_element_type=jnp.float32)
    h = h + b_ref[...]
    h_ref[...] = h
    hr_ref[...] = _ln_relu(h, g_ref[...], bb_ref[...])


def _mlp_update(h_ref, hr_ref, sa_ref, sb_ref, ca_ref, cb_ref,
                w1_ref, b1_ref, w2_ref, b2_ref):
    cnt = ca_ref[0][:, :1] + cb_ref[0][:, :1]
    s = sa_ref[0] + sb_ref[0]
    agg = (s + 1e-7 * cnt) / jnp.maximum(cnt, 1.0)
    u = hr_ref[...] + agg
    m = jnp.maximum(
        jnp.dot(u, w1_ref[...], preferred_element_type=jnp.float32)
        + b1_ref[...], 0.0)
    m = jnp.dot(m, w2_ref[...], preferred_element_type=jnp.float32) + b2_ref[...]
    return h_ref[...] + m


def _tc_mid_body(h_ref, hr_ref, sa_ref, sb_ref, ca_ref, cb_ref,
                 w1_ref, b1_ref, w2_ref, b2_ref,
                 g_ref, bb_ref, h1_ref, hr1_ref):
    h1 = _mlp_update(h_ref, hr_ref, sa_ref, sb_ref, ca_ref, cb_ref,
                     w1_ref, b1_ref, w2_ref, b2_ref)
    h1_ref[...] = h1
    hr1_ref[...] = _ln_relu(h1, g_ref[...], bb_ref[...])


def _tc_pool_body(h_ref, out_ref):
    out_ref[...] = (jnp.sum(h_ref[...], axis=0, keepdims=True) * (1.0 / BN))[None]


def _blk(shape, index_map):
    return pl.BlockSpec(shape, index_map)


_full0 = lambda i: (0, 0)
_rows = lambda i: (i, 0)

_tc_in = pl.pallas_call(
    _tc_in_body,
    grid=(G,),
    in_specs=[
        _blk((BN, D), _rows),
        _blk((D, D), _full0),
        _blk((1, D), _full0),
        _blk((1, D), _full0),
        _blk((1, D), _full0),
    ],
    out_specs=[_blk((BN, D), _rows), _blk((BN, D), _rows)],
    out_shape=[jax.ShapeDtypeStruct((N, D), jnp.float32)] * 2,
)

_sc0 = lambda i: (0, i, 0)
_sc1 = lambda i: (1, i, 0)

_mid_in_specs = [
    _blk((BN, D), _rows),
    _blk((BN, D), _rows),
    _blk((1, BN, D), _sc0),
    _blk((1, BN, D), _sc1),
    _blk((1, BN, D), _sc0),
    _blk((1, BN, D), _sc1),
    _blk((D, 2 * D), _full0),
    _blk((1, 2 * D), _full0),
    _blk((2 * D, D), _full0),
    _blk((1, D), _full0),
]

_tc_mid = pl.pallas_call(
    _tc_mid_body,
    grid=(G,),
    in_specs=_mid_in_specs + [_blk((1, D), _full0), _blk((1, D), _full0)],
    out_specs=[_blk((BN, D), _rows), _blk((BN, D), _rows)],
    out_shape=[jax.ShapeDtypeStruct((N, D), jnp.float32)] * 2,
)

_tc_pool = pl.pallas_call(
    _tc_pool_body,
    grid=(G,),
    in_specs=[_blk((BN, D), _rows)],
    out_specs=_blk((1, 1, D), lambda i: (i, 0, 0)),
    out_shape=jax.ShapeDtypeStruct((G, 1, D), jnp.float32),
)


def kernel(x, edge_index, num_graphs, W_in, b_in, Wm1_0, bm1_0, Wm2_0, bm2_0,
           ln_g_0, ln_b_0, Wm1_1, bm1_1, Wm2_1, bm2_1, ln_g_1, ln_b_1):
    pad = ((0, 0), (0, EPP - EPT))
    src_f = jnp.pad(edge_index[0].reshape(NW, EPT), pad).reshape(NW, NCF, K)
    dst = jnp.pad(edge_index[1].reshape(NW, EPT), pad,
                  constant_values=-1).reshape(1, NW * EPP)

    def remap(npass):
        nph = NP // npass
        p_arr = jnp.arange(npass, dtype=jnp.int32)[:, None]
        dl = dst - p_arr * nph
        ok = (dl >= 0) & (dl < nph)
        return jnp.where(ok, dl, nph + (dst & (TR - 1))).reshape(
            npass * NW, NCF, K)

    dst_c = remap(NPASS_C)
    dst_n = remap(NPASS_N)
    z_c = jnp.zeros(((NP // NPASS_C + TR) // NS, D), jnp.float32)
    z_n = jnp.zeros(((NP // NPASS_N + TR) // NS, D), jnp.float32)
    ones = jnp.ones((K, D), jnp.float32)

    sc_featsum_c, sc_featsum_n = _sc_kernels()
    h0, hr0 = _tc_in(x, W_in, b_in.reshape(1, D), ln_g_0.reshape(1, D),
                     ln_b_0.reshape(1, D))
    S0, C0 = sc_featsum_c(hr0, src_f, dst_c, z_c, ones)
    S0, C0 = S0.reshape(NC, NP, D), C0.reshape(NC, NP, D)
    h1, hr1 = _tc_mid(h0, hr0, S0, S0, C0, C0,
                      Wm1_0, bm1_0.reshape(1, 2 * D), Wm2_0,
                      bm2_0.reshape(1, D), ln_g_1.reshape(1, D),
                      ln_b_1.reshape(1, D))
    (S1,) = sc_featsum_n(hr1, src_f, dst_n, z_n, ones)
    S1 = S1.reshape(NC, NP, D)
    h2, _ = _tc_mid(h1, hr1, S1, S1, C0, C0,
                    Wm1_1, bm1_1.reshape(1, 2 * D), Wm2_1,
                    bm2_1.reshape(1, D), ln_g_1.reshape(1, D),
                    ln_b_1.reshape(1, D))
    return _tc_pool(h2).reshape(G, D)


# speedup vs baseline: 2.4731x; 1.5113x over previous
import functools

import jax
import jax.numpy as jnp
from jax import lax
from jax.experimental import pallas as pl
from jax.experimental.pallas import tpu as pltpu
from jax.experimental.pallas import tpu_sc as plsc

N = 10000
E = 320000
D = 128
G = 10
BN = 1000

NS = 16
NC = 2
NW = NC * NS
K = 128
EPT = E // NW
NCF = -(-EPT // K)
EPP = NCF * K
NP = 10240
NPASS_C = 4
NPASS_N = 4
TR = 128


@functools.cache
def _sc_kernels():
    mesh = plsc.VectorSubcoreMesh(core_axis_name="c", subcore_axis_name="s",
                                  num_cores=NC, num_subcores=NS)

    def _mk(with_counts, npass):
        nph = NP // npass
        npa = nph + TR
        npz = npa // NS
        npe = nph // NS
        out_type = [jax.ShapeDtypeStruct((NC * NP, D), jnp.float32)]
        scratch = [
            pltpu.VMEM((NCF, K), jnp.int32),
            pltpu.VMEM((NCF, K), jnp.int32),
            pltpu.VMEM((K, D), jnp.float32),
            pltpu.VMEM((npz, D), jnp.float32),
            pltpu.VMEM_SHARED((npa, D), jnp.float32),
        ]
        if with_counts:
            out_type.append(jax.ShapeDtypeStruct((NC * NP, D), jnp.float32))
            scratch.append(pltpu.VMEM((K, D), jnp.float32))
            scratch.append(pltpu.VMEM_SHARED((npa, D), jnp.float32))
        scratch.append(pltpu.SemaphoreType.DMA)

        @functools.partial(pl.kernel, out_type=tuple(out_type), mesh=mesh,
                           scratch_types=tuple(scratch))
        def sc_featsum(hr_hbm, src_hbm, dst_hbm, z_hbm, ones_hbm, *rest):
            if with_counts:
                (agg_out, cnt_out, src_v, dst_v, rows_v, z_v, agg_sh,
                 ones_v, cnt_sh, sem) = rest
            else:
                agg_out, src_v, dst_v, rows_v, z_v, agg_sh, sem = rest
            cid = lax.axis_index("c")
            sid = lax.axis_index("s")
            wid = cid * NS + sid
            pltpu.sync_copy(src_hbm.at[wid], src_v)
            pltpu.sync_copy(z_hbm, z_v)
            if with_counts:
                pltpu.sync_copy(ones_hbm, ones_v)
            for p in range(npass):
                pltpu.sync_copy(z_v, agg_sh.at[pl.ds(sid * npz, npz)])
                if with_counts:
                    pltpu.sync_copy(z_v, cnt_sh.at[pl.ds(sid * npz, npz)])
                pltpu.sync_copy(dst_hbm.at[p * NW + wid], dst_v)
                plsc.subcore_barrier()

                def body(j, carry):
                    cp = pltpu.async_copy(hr_hbm.at[src_v.at[j]], rows_v, sem)
                    cp.wait()
                    pltpu.sync_copy(rows_v, agg_sh.at[dst_v.at[j]], add=True)
                    if with_counts:
                        pltpu.sync_copy(ones_v, cnt_sh.at[dst_v.at[j]],
                                        add=True)
                    return carry

                lax.fori_loop(0, NCF, body, 0)
                plsc.subcore_barrier()
                pltpu.sync_copy(agg_sh.at[pl.ds(sid * npe, npe)],
                                z_v.at[pl.ds(0, npe)])
                pltpu.sync_copy(
                    z_v.at[pl.ds(0, npe)],
                    agg_out.at[pl.ds(cid * NP + p * nph + sid * npe, npe)])
                if with_counts:
                    pltpu.sync_copy(cnt_sh.at[pl.ds(sid * npe, npe)],
                                    z_v.at[pl.ds(0, npe)])
                    pltpu.sync_copy(
                        z_v.at[pl.ds(0, npe)],
                        cnt_out.at[pl.ds(cid * NP + p * nph + sid * npe,
                                         npe)])
                if p < npass - 1:
                    pltpu.sync_copy(z_hbm, z_v)
                    plsc.subcore_barrier()

        return sc_featsum

    return _mk(True, NPASS_C), _mk(False, NPASS_N)


def _ln_relu(h, g, b):
    mu = jnp.mean(h, axis=-1, keepdims=True)
    xc = h - mu
    var = jnp.mean(xc * xc, axis=-1, keepdims=True)
    hn = xc / jnp.sqrt(var + 1e-5) * g + b
    return jnp.maximum(hn, 0.0)


def _tc_in_body(x_ref, w_ref, b_ref, g_ref, bb_ref, h_ref, hr_ref):
    h = jnp.dot(x_ref[...], w_ref[...], preferred_element_type=jnp.float32)
    h = h + b_ref[...]
    h_ref[...] = h
    hr_ref[...] = _ln_relu(h, g_ref[...], bb_ref[...])


def _mlp_update(h_ref, hr_ref, sa_ref, sb_ref, ca_ref, cb_ref,
                w1_ref, b1_ref, w2_ref, b2_ref):
    cnt = ca_ref[0][:, :1] + cb_ref[0][:, :1]
    s = sa_ref[0] + sb_ref[0]
    agg = (s + 1e-7 * cnt) / jnp.maximum(cnt, 1.0)
    u = hr_ref[...] + agg
    m = jnp.maximum(
        jnp.dot(u, w1_ref[...], preferred_element_type=jnp.float32)
        + b1_ref[...], 0.0)
    m = jnp.dot(m, w2_ref[...], preferred_element_type=jnp.float32) + b2_ref[...]
    return h_ref[...] + m


def _tc_mid_body(h_ref, hr_ref, sa_ref, sb_ref, ca_ref, cb_ref,
                 w1_ref, b1_ref, w2_ref, b2_ref,
                 g_ref, bb_ref, h1_ref, hr1_ref):
    h1 = _mlp_update(h_ref, hr_ref, sa_ref, sb_ref, ca_ref, cb_ref,
                     w1_ref, b1_ref, w2_ref, b2_ref)
    h1_ref[...] = h1
    hr1_ref[...] = _ln_relu(h1, g_ref[...], bb_ref[...])


def _tc_pool_body(h_ref, out_ref):
    out_ref[...] = (jnp.sum(h_ref[...], axis=0, keepdims=True) * (1.0 / BN))[None]


def _blk(shape, index_map):
    return pl.BlockSpec(shape, index_map)


_full0 = lambda i: (0, 0)
_rows = lambda i: (i, 0)

_tc_in = pl.pallas_call(
    _tc_in_body,
    grid=(G,),
    in_specs=[
        _blk((BN, D), _rows),
        _blk((D, D), _full0),
        _blk((1, D), _full0),
        _blk((1, D), _full0),
        _blk((1, D), _full0),
    ],
    out_specs=[_blk((BN, D), _rows), _blk((BN, D), _rows)],
    out_shape=[jax.ShapeDtypeStruct((N, D), jnp.float32)] * 2,
)

_sc0 = lambda i: (0, i, 0)
_sc1 = lambda i: (1, i, 0)

_mid_in_specs = [
    _blk((BN, D), _rows),
    _blk((BN, D), _rows),
    _blk((1, BN, D), _sc0),
    _blk((1, BN, D), _sc1),
    _blk((1, BN, D), _sc0),
    _blk((1, BN, D), _sc1),
    _blk((D, 2 * D), _full0),
    _blk((1, 2 * D), _full0),
    _blk((2 * D, D), _full0),
    _blk((1, D), _full0),
]

_tc_mid = pl.pallas_call(
    _tc_mid_body,
    grid=(G,),
    in_specs=_mid_in_specs + [_blk((1, D), _full0), _blk((1, D), _full0)],
    out_specs=[_blk((BN, D), _rows), _blk((BN, D), _rows)],
    out_shape=[jax.ShapeDtypeStruct((N, D), jnp.float32)] * 2,
)

_tc_pool = pl.pallas_call(
    _tc_pool_body,
    grid=(G,),
    in_specs=[_blk((BN, D), _rows)],
    out_specs=_blk((1, 1, D), lambda i: (i, 0, 0)),
    out_shape=jax.ShapeDtypeStruct((G, 1, D), jnp.float32),
)


def kernel(x, edge_index, num_graphs, W_in, b_in, Wm1_0, bm1_0, Wm2_0, bm2_0,
           ln_g_0, ln_b_0, Wm1_1, bm1_1, Wm2_1, bm2_1, ln_g_1, ln_b_1):
    pad = ((0, 0), (0, EPP - EPT))
    src_f = jnp.pad(edge_index[0].reshape(NW, EPT), pad).reshape(NW, NCF, K)
    dst = jnp.pad(edge_index[1].reshape(NW, EPT), pad,
                  constant_values=-1).reshape(1, NW * EPP)

    def remap(npass):
        nph = NP // npass
        p_arr = jnp.arange(npass, dtype=jnp.int32)[:, None]
        dl = dst - p_arr * nph
        ok = (dl >= 0) & (dl < nph)
        return jnp.where(ok, dl, nph + (dst & (TR - 1))).reshape(
            npass * NW, NCF, K)

    dst_c = remap(NPASS_C)
    dst_n = remap(NPASS_N)
    z_c = jnp.zeros(((NP // NPASS_C + TR) // NS, D), jnp.float32)
    z_n = jnp.zeros(((NP // NPASS_N + TR) // NS, D), jnp.float32)
    ones = jnp.ones((K, D), jnp.float32)

    sc_featsum_c, sc_featsum_n = _sc_kernels()
    h0, hr0 = _tc_in(x, W_in, b_in.reshape(1, D), ln_g_0.reshape(1, D),
                     ln_b_0.reshape(1, D))
    S0, C0 = sc_featsum_c(hr0, src_f, dst_c, z_c, ones)
    S0, C0 = S0.reshape(NC, NP, D), C0.reshape(NC, NP, D)
    h1, hr1 = _tc_mid(h0, hr0, S0, S0, C0, C0,
                      Wm1_0, bm1_0.reshape(1, 2 * D), Wm2_0,
                      bm2_0.reshape(1, D), ln_g_1.reshape(1, D),
                      ln_b_1.reshape(1, D))
    (S1,) = sc_featsum_n(hr1, src_f, dst_n, z_n, ones)
    S1 = S1.reshape(NC, NP, D)
    h2, _ = _tc_mid(h1, hr1, S1, S1, C0, C0,
                    Wm1_1, bm1_1.reshape(1, 2 * D), Wm2_1,
                    bm2_1.reshape(1, D), ln_g_1.reshape(1, D),
                    ln_b_1.reshape(1, D))
    return _tc_pool(h2).reshape(G, D)


# speedup vs baseline: 3.1586x; 1.2772x over previous
import functools

import jax
import jax.numpy as jnp
from jax import lax
from jax.experimental import pallas as pl
from jax.experimental.pallas import tpu as pltpu
from jax.experimental.pallas import tpu_sc as plsc

N = 10000
E = 320000
D = 128
G = 10
BN = 1000

NS = 16
NC = 2
NW = NC * NS
K = 128
EPT = E // NW
NCF = -(-EPT // K)
EPP = NCF * K
NP = 10240
NPASS_C = 4
NPASS_N = 2
TR = 128


@functools.cache
def _sc_kernels():
    mesh = plsc.VectorSubcoreMesh(core_axis_name="c", subcore_axis_name="s",
                                  num_cores=NC, num_subcores=NS)

    def _mk(with_counts, npass):
        nph = NP // npass
        npa = nph + TR
        npz = npa // NS
        npe = nph // NS
        out_type = [jax.ShapeDtypeStruct((NC * NP, D), jnp.float32)]
        scratch = [
            pltpu.VMEM((NCF, K), jnp.int32),
            pltpu.VMEM((NCF, K), jnp.int32),
            pltpu.VMEM((K, D), jnp.float32),
            pltpu.VMEM((npz, D), jnp.float32),
            pltpu.VMEM_SHARED((npa, D), jnp.float32),
        ]
        if with_counts:
            out_type.append(jax.ShapeDtypeStruct((NC * NP, D), jnp.float32))
            scratch.append(pltpu.VMEM((K, D), jnp.float32))
            scratch.append(pltpu.VMEM_SHARED((npa, D), jnp.float32))
        scratch.append(pltpu.SemaphoreType.DMA)

        @functools.partial(pl.kernel, out_type=tuple(out_type), mesh=mesh,
                           scratch_types=tuple(scratch))
        def sc_featsum(hr_hbm, src_hbm, dst_hbm, z_hbm, ones_hbm, *rest):
            if with_counts:
                (agg_out, cnt_out, src_v, dst_v, rows_v, z_v, agg_sh,
                 ones_v, cnt_sh, sem) = rest
            else:
                agg_out, src_v, dst_v, rows_v, z_v, agg_sh, sem = rest
            cid = lax.axis_index("c")
            sid = lax.axis_index("s")
            wid = cid * NS + sid
            pltpu.sync_copy(src_hbm.at[wid], src_v)
            pltpu.sync_copy(z_hbm, z_v)
            if with_counts:
                pltpu.sync_copy(ones_hbm, ones_v)
            for p in range(npass):
                pltpu.sync_copy(z_v, agg_sh.at[pl.ds(sid * npz, npz)])
                if with_counts:
                    pltpu.sync_copy(z_v, cnt_sh.at[pl.ds(sid * npz, npz)])
                pltpu.sync_copy(dst_hbm.at[p * NW + wid], dst_v)
                plsc.subcore_barrier()

                def body(j, carry):
                    cp = pltpu.async_copy(hr_hbm.at[src_v.at[j]], rows_v, sem)
                    cp.wait()
                    pltpu.sync_copy(rows_v, agg_sh.at[dst_v.at[j]], add=True)
                    if with_counts:
                        pltpu.sync_copy(ones_v, cnt_sh.at[dst_v.at[j]],
                                        add=True)
                    return carry

                lax.fori_loop(0, NCF, body, 0)
                plsc.subcore_barrier()
                pltpu.sync_copy(agg_sh.at[pl.ds(sid * npe, npe)],
                                z_v.at[pl.ds(0, npe)])
                pltpu.sync_copy(
                    z_v.at[pl.ds(0, npe)],
                    agg_out.at[pl.ds(cid * NP + p * nph + sid * npe, npe)])
                if with_counts:
                    pltpu.sync_copy(cnt_sh.at[pl.ds(sid * npe, npe)],
                                    z_v.at[pl.ds(0, npe)])
                    pltpu.sync_copy(
                        z_v.at[pl.ds(0, npe)],
                        cnt_out.at[pl.ds(cid * NP + p * nph + sid * npe,
                                         npe)])
                if p < npass - 1:
                    pltpu.sync_copy(z_hbm, z_v)
                    plsc.subcore_barrier()

        return sc_featsum

    return _mk(True, NPASS_C), _mk(False, NPASS_N)


def _ln_relu(h, g, b):
    mu = jnp.mean(h, axis=-1, keepdims=True)
    xc = h - mu
    var = jnp.mean(xc * xc, axis=-1, keepdims=True)
    hn = xc / jnp.sqrt(var + 1e-5) * g + b
    return jnp.maximum(hn, 0.0)


def _tc_in_body(x_ref, w_ref, b_ref, g_ref, bb_ref, h_ref, hr_ref):
    h = jnp.dot(x_ref[...], w_ref[...], preferred_element_type=jnp.float32)
    h = h + b_ref[...]
    h_ref[...] = h
    hr_ref[...] = _ln_relu(h, g_ref[...], bb_ref[...])


def _mlp_update(h_ref, hr_ref, sa_ref, sb_ref, ca_ref, cb_ref,
                w1_ref, b1_ref, w2_ref, b2_ref):
    cnt = ca_ref[0][:, :1] + cb_ref[0][:, :1]
    s = sa_ref[0] + sb_ref[0]
    agg = (s + 1e-7 * cnt) / jnp.maximum(cnt, 1.0)
    u = hr_ref[...] + agg
    m = jnp.maximum(
        jnp.dot(u, w1_ref[...], preferred_element_type=jnp.float32)
        + b1_ref[...], 0.0)
    m = jnp.dot(m, w2_ref[...], preferred_element_type=jnp.float32) + b2_ref[...]
    return h_ref[...] + m


def _tc_mid_body(h_ref, hr_ref, sa_ref, sb_ref, ca_ref, cb_ref,
                 w1_ref, b1_ref, w2_ref, b2_ref,
                 g_ref, bb_ref, h1_ref, hr1_ref):
    h1 = _mlp_update(h_ref, hr_ref, sa_ref, sb_ref, ca_ref, cb_ref,
                     w1_ref, b1_ref, w2_ref, b2_ref)
    h1_ref[...] = h1
    hr1_ref[...] = _ln_relu(h1, g_ref[...], bb_ref[...])


def _tc_pool_body(h_ref, out_ref):
    out_ref[...] = (jnp.sum(h_ref[...], axis=0, keepdims=True) * (1.0 / BN))[None]


def _blk(shape, index_map):
    return pl.BlockSpec(shape, index_map)


_full0 = lambda i: (0, 0)
_rows = lambda i: (i, 0)

_tc_in = pl.pallas_call(
    _tc_in_body,
    grid=(G,),
    in_specs=[
        _blk((BN, D), _rows),
        _blk((D, D), _full0),
        _blk((1, D), _full0),
        _blk((1, D), _full0),
        _blk((1, D), _full0),
    ],
    out_specs=[_blk((BN, D), _rows), _blk((BN, D), _rows)],
    out_shape=[jax.ShapeDtypeStruct((N, D), jnp.float32)] * 2,
)

_sc0 = lambda i: (0, i, 0)
_sc1 = lambda i: (1, i, 0)

_mid_in_specs = [
    _blk((BN, D), _rows),
    _blk((BN, D), _rows),
    _blk((1, BN, D), _sc0),
    _blk((1, BN, D), _sc1),
    _blk((1, BN, D), _sc0),
    _blk((1, BN, D), _sc1),
    _blk((D, 2 * D), _full0),
    _blk((1, 2 * D), _full0),
    _blk((2 * D, D), _full0),
    _blk((1, D), _full0),
]

_tc_mid = pl.pallas_call(
    _tc_mid_body,
    grid=(G,),
    in_specs=_mid_in_specs + [_blk((1, D), _full0), _blk((1, D), _full0)],
    out_specs=[_blk((BN, D), _rows), _blk((BN, D), _rows)],
    out_shape=[jax.ShapeDtypeStruct((N, D), jnp.float32)] * 2,
)

_tc_pool = pl.pallas_call(
    _tc_pool_body,
    grid=(G,),
    in_specs=[_blk((BN, D), _rows)],
    out_specs=_blk((1, 1, D), lambda i: (i, 0, 0)),
    out_shape=jax.ShapeDtypeStruct((G, 1, D), jnp.float32),
)


def kernel(x, edge_index, num_graphs, W_in, b_in, Wm1_0, bm1_0, Wm2_0, bm2_0,
           ln_g_0, ln_b_0, Wm1_1, bm1_1, Wm2_1, bm2_1, ln_g_1, ln_b_1):
    pad = ((0, 0), (0, EPP - EPT))
    src_f = jnp.pad(edge_index[0].reshape(NW, EPT), pad).reshape(NW, NCF, K)
    dst = jnp.pad(edge_index[1].reshape(NW, EPT), pad,
                  constant_values=-1).reshape(1, NW * EPP)

    def remap(npass):
        nph = NP // npass
        p_arr = jnp.arange(npass, dtype=jnp.int32)[:, None]
        dl = dst - p_arr * nph
        ok = (dl >= 0) & (dl < nph)
        return jnp.where(ok, dl, nph + (dst & (TR - 1))).reshape(
            npass * NW, NCF, K)

    dst_c = remap(NPASS_C)
    dst_n = remap(NPASS_N)
    z_c = jnp.zeros(((NP // NPASS_C + TR) // NS, D), jnp.float32)
    z_n = jnp.zeros(((NP // NPASS_N + TR) // NS, D), jnp.float32)
    ones = jnp.ones((K, D), jnp.float32)

    sc_featsum_c, sc_featsum_n = _sc_kernels()
    h0, hr0 = _tc_in(x, W_in, b_in.reshape(1, D), ln_g_0.reshape(1, D),
                     ln_b_0.reshape(1, D))
    S0, C0 = sc_featsum_c(hr0, src_f, dst_c, z_c, ones)
    S0, C0 = S0.reshape(NC, NP, D), C0.reshape(NC, NP, D)
    h1, hr1 = _tc_mid(h0, hr0, S0, S0, C0, C0,
                      Wm1_0, bm1_0.reshape(1, 2 * D), Wm2_0,
                      bm2_0.reshape(1, D), ln_g_1.reshape(1, D),
                      ln_b_1.reshape(1, D))
    (S1,) = sc_featsum_n(hr1, src_f, dst_n, z_n, ones)
    S1 = S1.reshape(NC, NP, D)
    h2, _ = _tc_mid(h1, hr1, S1, S1, C0, C0,
                    Wm1_1, bm1_1.reshape(1, 2 * D), Wm2_1,
                    bm2_1.reshape(1, D), ln_g_1.reshape(1, D),
                    ln_b_1.reshape(1, D))
    return _tc_pool(h2).reshape(G, D)
